# baseline (speedup 1.0000x reference)
import functools

import jax
import jax.numpy as jnp
from jax import lax
from jax.experimental import pallas as pl
from jax.experimental.pallas import tpu as pltpu
from jax.experimental.pallas import tpu_sc as plsc

N = 10000
E = 320000
D = 128

NC = 2
NS = 16
NW = NC * NS
EPW = E // NW
CHUNK = 80
NCHUNK = EPW // CHUNK
CHUNK_A = 40
NCHUNK_A = EPW // CHUNK_A
NPAD = 10112
RPW = NPAD // NS
DEGW = 128

_MESH = plsc.VectorSubcoreMesh(
    core_axis_name="c", subcore_axis_name="s", num_cores=NC, num_subcores=NS)


NBUF = 5


def _deg_body(dst3, ones_hbm, zeros_hbm, out_hbm, deg_sh, didx_v, ones_v, sem):
  cid = lax.axis_index("c")
  sid = lax.axis_index("s")
  wid = sid * NC + cid
  pltpu.sync_copy(zeros_hbm.at[pl.ds(sid * RPW, RPW)],
                  deg_sh.at[pl.ds(sid * RPW, RPW)])
  pltpu.sync_copy(ones_hbm, ones_v)
  pltpu.sync_copy(dst3.at[wid], didx_v)
  plsc.subcore_barrier()

  def group(g, carry):
    base = g * NBUF
    for b in range(NBUF):
      pltpu.async_copy(ones_v, deg_sh.at[didx_v.at[base + b]], sem, add=True)
    for b in range(NBUF):
      pltpu.make_async_copy(ones_hbm, ones_v, sem).wait()
    return carry

  lax.fori_loop(0, NCHUNK // NBUF, group, 0)
  plsc.subcore_barrier()
  pltpu.sync_copy(deg_sh.at[pl.ds(sid * RPW, RPW)],
                  out_hbm.at[cid, pl.ds(sid * RPW, RPW)])


_deg_kernel = pl.kernel(
    _deg_body,
    out_type=jax.ShapeDtypeStruct((NC, NPAD, DEGW), jnp.float32),
    mesh=_MESH,
    scratch_types=[
        pltpu.VMEM_SHARED((NPAD, DEGW), jnp.float32),
        pltpu.VMEM((NCHUNK, CHUNK), jnp.int32),
        pltpu.VMEM((CHUNK, DEGW), jnp.float32),
        pltpu.SemaphoreType.DMA,
    ],
)


def _agg_body(sd3, xs_hbm, zeros_hbm, out_hbm, s_sh,
              sd_v, rows0_v, rows1_v, rows2_v,
              sem0, sem1, sem2, ssem0, ssem1, ssem2):
  cid = lax.axis_index("c")
  sid = lax.axis_index("s")
  wid = sid * NC + cid
  pltpu.sync_copy(zeros_hbm.at[pl.ds(sid * RPW, RPW)],
                  s_sh.at[pl.ds(sid * RPW, RPW)])
  pltpu.sync_copy(sd3.at[wid], sd_v)
  plsc.subcore_barrier()

  def sidx(k):
    return sd_v.at[k, pl.ds(0, CHUNK_A)]

  def didx(k):
    return sd_v.at[k, pl.ds(CHUNK_A, CHUNK_A)]

  bufs = (rows0_v, rows1_v, rows2_v)
  gsems = (sem0, sem1, sem2)
  ssems = (ssem0, ssem1, ssem2)

  def fg(k, b):
    pltpu.async_copy(xs_hbm.at[sidx(k)], bufs[b], gsems[b])

  def wg(b):
    pltpu.make_async_copy(xs_hbm.at[pl.ds(0, CHUNK_A)], bufs[b],
                          gsems[b]).wait()

  def fs(k, b):
    pltpu.async_copy(bufs[b], s_sh.at[didx(k)], ssems[b], add=True)

  def ws(b):
    pltpu.make_async_copy(xs_hbm.at[pl.ds(0, CHUNK_A)], bufs[b],
                          ssems[b]).wait()

  fg(0, 0)
  fg(1, 1)
  wg(0)
  fs(0, 0)
  fg(2, 2)
  wg(1)
  fs(1, 1)
  ws(0)
  fg(3, 0)

  def step(k, b):
    bn = (b + 2) % 3
    wg(b)
    fs(k, b)
    ws(bn)
    fg(k + 2, bn)

  def step3(g, carry):
    k0 = 3 * g + 2
    step(k0, 2)
    step(k0 + 1, 0)
    step(k0 + 2, 1)
    return carry

  lax.fori_loop(0, (NCHUNK_A - 4) // 3, step3, 0)
  wg(2)
  fs(NCHUNK_A - 2, 2)
  wg(0)
  fs(NCHUNK_A - 1, 0)
  ws(0)
  ws(1)
  ws(2)
  plsc.subcore_barrier()
  pltpu.sync_copy(s_sh.at[pl.ds(sid * RPW, RPW)],
                  out_hbm.at[cid, pl.ds(sid * RPW, RPW)])


_agg_kernel = pl.kernel(
    _agg_body,
    out_type=jax.ShapeDtypeStruct((NC, NPAD, D), jnp.float32),
    mesh=_MESH,
    scratch_types=[
        pltpu.VMEM_SHARED((NPAD, D), jnp.float32),
        pltpu.VMEM((NCHUNK_A, 2 * CHUNK_A), jnp.int32),
        pltpu.VMEM((CHUNK_A, D), jnp.float32),
        pltpu.VMEM((CHUNK_A, D), jnp.float32),
        pltpu.VMEM((CHUNK_A, D), jnp.float32),
        pltpu.SemaphoreType.DMA,
        pltpu.SemaphoreType.DMA,
        pltpu.SemaphoreType.DMA,
        pltpu.SemaphoreType.DMA,
        pltpu.SemaphoreType.DMA,
        pltpu.SemaphoreType.DMA,
    ],
)


def _scale_body(degp_ref, x_ref, xs_ref, dinv_ref):
  deg = degp_ref[0, :, 0] + degp_ref[1, :, 0] + 1.0
  dinv = lax.rsqrt(deg)
  xs_ref[...] = x_ref[...] * dinv[:, None]
  dinv_ref[...] = jnp.broadcast_to(dinv[:, None], dinv_ref.shape)


def _dense_body(s_ref, xs_ref, dinv_ref, w2_ref, b2_ref, wo_ref, bo_ref,
                out_ref):
  t = s_ref[0] + s_ref[1] + xs_ref[...]
  agg = t * dinv_ref[:, :1]
  z = jnp.dot(agg, w2_ref[...], preferred_element_type=jnp.float32)
  z = jnp.maximum(z + b2_ref[...], 0.0)
  out_ref[...] = (
      jnp.dot(z, wo_ref[...], preferred_element_type=jnp.float32)
      + bo_ref[...])


_BLK = 1000


def _tc_scale(degp, x):
  return pl.pallas_call(
      _scale_body,
      grid=(N // _BLK,),
      in_specs=[
          pl.BlockSpec((NC, _BLK, DEGW), lambda i: (0, i, 0)),
          pl.BlockSpec((_BLK, D), lambda i: (i, 0)),
      ],
      out_specs=[
          pl.BlockSpec((_BLK, D), lambda i: (i, 0)),
          pl.BlockSpec((_BLK, DEGW), lambda i: (i, 0)),
      ],
      out_shape=[
          jax.ShapeDtypeStruct((N, D), jnp.float32),
          jax.ShapeDtypeStruct((N, DEGW), jnp.float32),
      ],
  )(degp, x)


def _tc_dense(s, xs, dinv, W2, b2, Wo, bo):
  c = Wo.shape[1]
  return pl.pallas_call(
      _dense_body,
      grid=(N // _BLK,),
      in_specs=[
          pl.BlockSpec((NC, _BLK, D), lambda i: (0, i, 0)),
          pl.BlockSpec((_BLK, D), lambda i: (i, 0)),
          pl.BlockSpec((_BLK, DEGW), lambda i: (i, 0)),
          pl.BlockSpec((D, D), lambda i: (0, 0)),
          pl.BlockSpec((1, D), lambda i: (0, 0)),
          pl.BlockSpec((D, c), lambda i: (0, 0)),
          pl.BlockSpec((1, c), lambda i: (0, 0)),
      ],
      out_specs=pl.BlockSpec((_BLK, c), lambda i: (i, 0)),
      out_shape=jax.ShapeDtypeStruct((N, c), jnp.float32),
  )(s, xs, dinv, W2, b2.reshape(1, D), Wo, bo.reshape(1, c))


@jax.jit
def kernel(x_content, edge_index, edge_type, W1, b1, W2, b2, Wo, bo):
  del edge_type, W1, b1
  src3 = edge_index[0].reshape(NW, NCHUNK, CHUNK)
  dst3 = edge_index[1].reshape(NW, NCHUNK, CHUNK)
  sd3 = jnp.concatenate(
      [edge_index[0].reshape(NW, NCHUNK_A, CHUNK_A),
       edge_index[1].reshape(NW, NCHUNK_A, CHUNK_A)], axis=-1)
  ones_rows = jnp.ones((CHUNK, DEGW), jnp.float32)
  zeros_deg = jnp.zeros((NPAD, DEGW), jnp.float32)
  zeros_rows = jnp.zeros((NPAD, D), jnp.float32)

  degp = _deg_kernel(dst3, ones_rows, zeros_deg)
  xs, dinv = _tc_scale(degp, x_content)
  s = _agg_kernel(sd3, xs, zeros_rows)
  return _tc_dense(s, xs, dinv, W2, b2, Wo, bo)


# speedup vs baseline: 1.1341x; 1.1341x over previous
import functools

import jax
import jax.numpy as jnp
from jax import lax
from jax.experimental import pallas as pl
from jax.experimental.pallas import tpu as pltpu
from jax.experimental.pallas import tpu_sc as plsc

N = 10000
E = 320000
D = 128

NC = 2
NS = 16
NW = NC * NS
EPW = E // NW
CHUNK = 80
NCHUNK = EPW // CHUNK
CHUNK_A = 40
NCHUNK_A = EPW // CHUNK_A
NPAD = 10112
RPW = NPAD // NS
DEGW = 128

_MESH = plsc.VectorSubcoreMesh(
    core_axis_name="c", subcore_axis_name="s", num_cores=NC, num_subcores=NS)


NBUF = 5


def _deg_body(dst3, ones_hbm, zeros_hbm, out_hbm, deg_sh, didx_v, ones_v, sem):
  cid = lax.axis_index("c")
  sid = lax.axis_index("s")
  wid = sid * NC + cid
  pltpu.sync_copy(zeros_hbm.at[pl.ds(sid * RPW, RPW)],
                  deg_sh.at[pl.ds(sid * RPW, RPW)])
  pltpu.sync_copy(ones_hbm, ones_v)
  pltpu.sync_copy(dst3.at[wid], didx_v)
  plsc.subcore_barrier()

  def group(g, carry):
    base = g * NBUF
    for b in range(NBUF):
      pltpu.async_copy(ones_v, deg_sh.at[didx_v.at[base + b]], sem, add=True)
    for b in range(NBUF):
      pltpu.make_async_copy(ones_hbm, ones_v, sem).wait()
    return carry

  lax.fori_loop(0, NCHUNK // NBUF, group, 0)
  plsc.subcore_barrier()
  pltpu.sync_copy(deg_sh.at[pl.ds(sid * RPW, RPW)],
                  out_hbm.at[cid, pl.ds(sid * RPW, RPW)])


_deg_kernel = pl.kernel(
    _deg_body,
    out_type=jax.ShapeDtypeStruct((NC, NPAD, DEGW), jnp.float32),
    mesh=_MESH,
    scratch_types=[
        pltpu.VMEM_SHARED((NPAD, DEGW), jnp.float32),
        pltpu.VMEM((NCHUNK, CHUNK), jnp.int32),
        pltpu.VMEM((CHUNK, DEGW), jnp.float32),
        pltpu.SemaphoreType.DMA,
    ],
)


NRING = 5
NGRP = 5
GCHUNK = NCHUNK_A // NGRP


def _agg_body(sd3, xs_hbm, zeros_hbm, out_hbm, s_sh,
              idxa_v, idxb_v, r0, r1, r2, r3, r4,
              sem0, sem1, sem2, sem3, sem4, isem):
  cid = lax.axis_index("c")
  sid = lax.axis_index("s")
  wid = sid * NC + cid
  pltpu.sync_copy(zeros_hbm.at[pl.ds(sid * RPW, RPW)],
                  s_sh.at[pl.ds(sid * RPW, RPW)])
  pltpu.sync_copy(sd3.at[wid, 0], idxa_v)
  plsc.subcore_barrier()

  bufs = (r0, r1, r2, r3, r4)
  sems = (sem0, sem1, sem2, sem3, sem4)
  idxbufs = (idxa_v, idxb_v)

  for g in range(NGRP):
    idx_v = idxbufs[g % 2]
    if g > 0:
      pltpu.make_async_copy(sd3.at[wid, 0], idx_v, isem).wait()
    if g < NGRP - 1:
      pltpu.async_copy(sd3.at[wid, g + 1], idxbufs[(g + 1) % 2], isem)

    def fire(k, b, idx_v=idx_v):
      pltpu.async_copy(xs_hbm.at[idx_v.at[k, pl.ds(0, CHUNK_A)]],
                       bufs[b], sems[b])

    def scat(k, b, idx_v=idx_v):
      pltpu.make_async_copy(xs_hbm.at[pl.ds(0, CHUNK_A)], bufs[b],
                            sems[b]).wait()
      pltpu.sync_copy(bufs[b],
                      s_sh.at[idx_v.at[k, pl.ds(CHUNK_A, CHUNK_A)]], add=True)

    for b in range(NRING - 1):
      fire(b, b)

    def step5(j, carry):
      k0 = NRING * j
      for u in range(NRING):
        fire(k0 + u + NRING - 1, (u + NRING - 1) % NRING)
        scat(k0 + u, u)
      return carry

    lax.fori_loop(0, GCHUNK // NRING - 1, step5, 0)
    fire(GCHUNK - 1, (GCHUNK - 1) % NRING)
    for u in range(NRING):
      scat(GCHUNK - NRING + u, (GCHUNK - NRING + u) % NRING)

  plsc.subcore_barrier()
  pltpu.sync_copy(s_sh.at[pl.ds(sid * RPW, RPW)],
                  out_hbm.at[cid, pl.ds(sid * RPW, RPW)])


_agg_kernel = pl.kernel(
    _agg_body,
    out_type=jax.ShapeDtypeStruct((NC, NPAD, D), jnp.float32),
    mesh=_MESH,
    scratch_types=[
        pltpu.VMEM_SHARED((NPAD, D), jnp.float32),
        pltpu.VMEM((GCHUNK, 2 * CHUNK_A), jnp.int32),
        pltpu.VMEM((GCHUNK, 2 * CHUNK_A), jnp.int32),
        pltpu.VMEM((CHUNK_A, D), jnp.float32),
        pltpu.VMEM((CHUNK_A, D), jnp.float32),
        pltpu.VMEM((CHUNK_A, D), jnp.float32),
        pltpu.VMEM((CHUNK_A, D), jnp.float32),
        pltpu.VMEM((CHUNK_A, D), jnp.float32),
        pltpu.SemaphoreType.DMA,
        pltpu.SemaphoreType.DMA,
        pltpu.SemaphoreType.DMA,
        pltpu.SemaphoreType.DMA,
        pltpu.SemaphoreType.DMA,
        pltpu.SemaphoreType.DMA,
    ],
)


def _scale_body(degp_ref, x_ref, xs_ref, dinv_ref):
  deg = degp_ref[0, :, 0] + degp_ref[1, :, 0] + 1.0
  dinv = lax.rsqrt(deg)
  xs_ref[...] = x_ref[...] * dinv[:, None]
  dinv_ref[...] = jnp.broadcast_to(dinv[:, None], dinv_ref.shape)


def _dense_body(s_ref, xs_ref, dinv_ref, w2_ref, b2_ref, wo_ref, bo_ref,
                out_ref):
  t = s_ref[0] + s_ref[1] + xs_ref[...]
  agg = t * dinv_ref[:, :1]
  z = jnp.dot(agg, w2_ref[...], preferred_element_type=jnp.float32)
  z = jnp.maximum(z + b2_ref[...], 0.0)
  out_ref[...] = (
      jnp.dot(z, wo_ref[...], preferred_element_type=jnp.float32)
      + bo_ref[...])


_BLK = 1000


def _tc_scale(degp, x):
  return pl.pallas_call(
      _scale_body,
      grid=(N // _BLK,),
      in_specs=[
          pl.BlockSpec((NC, _BLK, DEGW), lambda i: (0, i, 0)),
          pl.BlockSpec((_BLK, D), lambda i: (i, 0)),
      ],
      out_specs=[
          pl.BlockSpec((_BLK, D), lambda i: (i, 0)),
          pl.BlockSpec((_BLK, DEGW), lambda i: (i, 0)),
      ],
      out_shape=[
          jax.ShapeDtypeStruct((N, D), jnp.float32),
          jax.ShapeDtypeStruct((N, DEGW), jnp.float32),
      ],
  )(degp, x)


def _tc_dense(s, xs, dinv, W2, b2, Wo, bo):
  c = Wo.shape[1]
  return pl.pallas_call(
      _dense_body,
      grid=(N // _BLK,),
      in_specs=[
          pl.BlockSpec((NC, _BLK, D), lambda i: (0, i, 0)),
          pl.BlockSpec((_BLK, D), lambda i: (i, 0)),
          pl.BlockSpec((_BLK, DEGW), lambda i: (i, 0)),
          pl.BlockSpec((D, D), lambda i: (0, 0)),
          pl.BlockSpec((1, D), lambda i: (0, 0)),
          pl.BlockSpec((D, c), lambda i: (0, 0)),
          pl.BlockSpec((1, c), lambda i: (0, 0)),
      ],
      out_specs=pl.BlockSpec((_BLK, c), lambda i: (i, 0)),
      out_shape=jax.ShapeDtypeStruct((N, c), jnp.float32),
  )(s, xs, dinv, W2, b2.reshape(1, D), Wo, bo.reshape(1, c))


@jax.jit
def kernel(x_content, edge_index, edge_type, W1, b1, W2, b2, Wo, bo):
  del edge_type, W1, b1
  src3 = edge_index[0].reshape(NW, NCHUNK, CHUNK)
  dst3 = edge_index[1].reshape(NW, NCHUNK, CHUNK)
  sd3 = jnp.concatenate(
      [edge_index[0].reshape(NW, NCHUNK_A, CHUNK_A),
       edge_index[1].reshape(NW, NCHUNK_A, CHUNK_A)],
      axis=-1).reshape(NW, NGRP, GCHUNK, 2 * CHUNK_A)
  ones_rows = jnp.ones((CHUNK, DEGW), jnp.float32)
  zeros_deg = jnp.zeros((NPAD, DEGW), jnp.float32)
  zeros_rows = jnp.zeros((NPAD, D), jnp.float32)

  degp = _deg_kernel(dst3, ones_rows, zeros_deg)
  xs, dinv = _tc_scale(degp, x_content)
  s = _agg_kernel(sd3, xs, zeros_rows)
  return _tc_dense(s, xs, dinv, W2, b2, Wo, bo)
